# Initial kernel scaffold; baseline (speedup 1.0000x reference)
import jax
import jax.numpy as jnp
from jax.experimental import pallas as pl


def kernel(coords, atoms, masses, edge_index, batch_ptrs, moments, t, params):
    raise NotImplementedError("write your pallas kernel here")


# speedup vs baseline: 1.4011x; 1.4011x over previous
import functools
import numpy as np
import jax
import jax.numpy as jnp
from jax import lax
from jax.experimental import pallas as pl
from jax.experimental.pallas import tpu as pltpu
from jax.experimental.pallas import tpu_sc as plsc

N = 10000; E = 320000; B = 100; F = 64; C = 64; EF = 16; H = 8; NL = 3

NC, NS = 2, 16
NW = NC * NS
CE = E // NW
BE = 80
TD = 80
TN = 2000
TE = 4000

_f32 = jnp.float32


def _mm(a, b):
    return jnp.dot(a, b, preferred_element_type=_f32,
                   precision=lax.Precision.HIGHEST)


def _silu(x):
    return x * (1.0 / (1.0 + jnp.exp(-x)))


def _sigmoid(x):
    return 1.0 / (1.0 + jnp.exp(-x))


def _full(shape):
    nd = len(shape)
    return pl.BlockSpec(shape, lambda i, _nd=nd: (0,) * _nd)


def _cond_body(t_ref, mom_ref, st_ref, sm_ref, w1_ref, b1_ref, w2_ref,
               b2_ref, y_ref):
    t = t_ref[...]
    mom = mom_ref[...]
    st = st_ref[...]
    sm = sm_ref[...]
    ang_t = t * st
    parts = [jnp.sin(ang_t), jnp.cos(ang_t)]
    for j in range(3):
        ang = mom[:, j:j + 1] * sm
        parts.append(jnp.sin(ang))
        parts.append(jnp.cos(ang))
    parts.append(mom)
    x = jnp.concatenate(parts, axis=1)
    h = _silu(_mm(x, w1_ref[...]) + b1_ref[...])
    y_ref[...] = _silu(_mm(h, w2_ref[...]) + b2_ref[...])


def _hembed_body(coords_ref, atoms_ref, mass_ref, emb_ref, w1_ref, b1_ref,
                 w2_ref, b2_ref, h_ref):
    at = atoms_ref[...]
    oh = (lax.broadcasted_iota(jnp.int32, (TN, 90), 1) == at).astype(_f32)
    aemb = _mm(oh, emb_ref[...])
    x = jnp.concatenate([jnp.abs(coords_ref[...]), aemb, mass_ref[...]],
                        axis=1)
    hh = _silu(_mm(x, w1_ref[...]) + b1_ref[...])
    h_ref[...] = _mm(hh, w2_ref[...]) + b2_ref[...]


def _ada_ln(h, y, ptr, wc, bc, pid):
    gb = _mm(y, wc) + bc
    gid = pid * TN + lax.broadcasted_iota(jnp.int32, (TN, B), 0)
    lo = ptr[:, :B]
    hi = ptr[:, 1:]
    P = ((gid >= lo) & (gid < hi)).astype(_f32)
    gbn = _mm(P, gb)
    mu = jnp.mean(h, axis=1, keepdims=True)
    d = h - mu
    var = jnp.mean(d * d, axis=1, keepdims=True)
    hn = d * lax.rsqrt(var + 1e-6)
    return hn * (1.0 + gbn[:, :F]) + gbn[:, F:]


def _make_blocknode_body(with_coords):
    def body(h_ref, y_ref, ptr_ref, coords_ref, wc_ref, bc_ref, wqs_ref,
             wqd_ref, sel3_ref, ts_ref, td_ref, hn_ref):
        hn = _ada_ln(h_ref[...], y_ref[...], ptr_ref[...], wc_ref[...],
                     bc_ref[...], pl.program_id(0))
        ts = _mm(hn, wqs_ref[...])
        td = _mm(hn, wqd_ref[...])
        if with_coords:
            cexp = _mm(coords_ref[...], sel3_ref[...])
            ts = ts + cexp
            td = td - cexp
        ts_ref[...] = ts
        td_ref[...] = td
        hn_ref[...] = hn
    return body


def _make_edge_body(first, last):
    def body(es_ref, ed_ref, a_ref, wma_ref, bm_ref, wga_ref, bg_ref,
             rexp_ref, we1_ref, be1_ref, we2_ref, be2_ref, wa_ref, ba_ref,
             m_ref, anew_ref):
        e = es_ref[...] + ed_ref[...]
        if first:
            dxyz = jnp.abs(e[:, 72:75])
            ah = _silu(_mm(dxyz, we1_ref[...]) + be1_ref[...])
            a = _mm(ah, we2_ref[...]) + be2_ref[...]
        else:
            a = a_ref[...]
        pm = e[:, :F] + _mm(a, wma_ref[...]) + bm_ref[...]
        pg = e[:, F:F + H] + _mm(a, wga_ref[...]) + bg_ref[...]
        m = _silu(pm) * _mm(_sigmoid(pg), rexp_ref[...])
        m_ref[...] = m
        if not last:
            anew_ref[...] = a + _mm(m, wa_ref[...]) + ba_ref[...]
    return body


def _update_body(h_ref, hn_ref, p0_ref, p1_ref, wu1h_ref, wu1a_ref, bu1_ref,
                 wu2_ref, bu2_ref, ho_ref):
    agg = p0_ref[...] + p1_ref[...]
    u = _silu(_mm(hn_ref[...], wu1h_ref[...]) + _mm(agg, wu1a_ref[...])
              + bu1_ref[...])
    ho_ref[...] = h_ref[...] + _mm(u, wu2_ref[...]) + bu2_ref[...]


def _head_body(h_ref, y_ref, ptr_ref, coords_ref, wc_ref, bc_ref, w1_ref,
               b1_ref, w2_ref, b2_ref, o_ref):
    hn = _ada_ln(h_ref[...], y_ref[...], ptr_ref[...], wc_ref[...],
                 bc_ref[...], pl.program_id(0))
    f = _silu(_mm(hn, w1_ref[...]) + b1_ref[...])
    o_ref[...] = jnp.sign(coords_ref[...]) * (_mm(f, w2_ref[...]) + b2_ref[...])


@functools.cache
def _sc_kernels():
    mesh = plsc.VectorSubcoreMesh(core_axis_name="c", subcore_axis_name="s")

    @functools.partial(
        pl.kernel,
        out_type=(jax.ShapeDtypeStruct((E, TD), _f32),
                  jax.ShapeDtypeStruct((E, TD), _f32)),
        mesh=mesh,
        scratch_types=[
            pltpu.VMEM((BE,), jnp.int32),
            pltpu.VMEM((BE,), jnp.int32),
            pltpu.VMEM((BE, TD), _f32),
            pltpu.VMEM((BE, TD), _f32),
            pltpu.SemaphoreType.DMA,
            pltpu.SemaphoreType.DMA,
        ],
        compiler_params=pltpu.CompilerParams(use_tc_tiling_on_sc=False),
    )
    def sc_gather(ts_hbm, td_hbm, src_hbm, dst_hbm, os_hbm, od_hbm,
                  idx_s, idx_d, rows_s, rows_d, sem_s, sem_d):
        wid = lax.axis_index("s") * NC + lax.axis_index("c")
        base = wid * CE

        def step(j, carry):
            off = base + j * BE
            pltpu.sync_copy(src_hbm.at[pl.ds(off, BE)], idx_s)
            pltpu.sync_copy(dst_hbm.at[pl.ds(off, BE)], idx_d)
            cp1 = pltpu.async_copy(ts_hbm.at[idx_s], rows_s, sem_s)
            cp2 = pltpu.async_copy(td_hbm.at[idx_d], rows_d, sem_d)
            cp1.wait()
            cp2.wait()
            pltpu.sync_copy(rows_s, os_hbm.at[pl.ds(off, BE)])
            pltpu.sync_copy(rows_d, od_hbm.at[pl.ds(off, BE)])
            return carry

        lax.fori_loop(0, CE // BE, step, 0)

    @functools.partial(
        pl.kernel,
        out_type=jax.ShapeDtypeStruct((NC, N, F), _f32),
        mesh=mesh,
        scratch_types=[
            pltpu.VMEM((BE,), jnp.int32),
            pltpu.VMEM((BE, F), _f32),
            pltpu.VMEM_SHARED((N, F), _f32),
        ],
        compiler_params=pltpu.CompilerParams(use_tc_tiling_on_sc=False),
    )
    def sc_scatter(m_hbm, dst_hbm, zero_hbm, out_hbm, idxb, mb, acc):
        c = lax.axis_index("c")
        s = lax.axis_index("s")
        wid = s * NC + c
        rows_per = N // NS
        pltpu.sync_copy(zero_hbm.at[pl.ds(s * rows_per, rows_per)],
                        acc.at[pl.ds(s * rows_per, rows_per)])
        plsc.subcore_barrier()
        base = wid * CE

        def step(j, carry):
            off = base + j * BE
            pltpu.sync_copy(dst_hbm.at[pl.ds(off, BE)], idxb)
            pltpu.sync_copy(m_hbm.at[pl.ds(off, BE)], mb)
            pltpu.sync_copy(mb, acc.at[idxb], add=True)
            return carry

        lax.fori_loop(0, CE // BE, step, 0)
        plsc.subcore_barrier()
        pltpu.sync_copy(acc.at[pl.ds(s * rows_per, rows_per)],
                        out_hbm.at[c, pl.ds(s * rows_per, rows_per)])

    return sc_gather, sc_scatter


def _grid_call(body, grid, in_arrays, in_specs, out_shapes, out_specs):
    return pl.pallas_call(
        body, grid=(grid,), in_specs=in_specs, out_specs=out_specs,
        out_shape=out_shapes)(*in_arrays)


def kernel(coords, atoms, masses, edge_index, batch_ptrs, moments, t, params):
    p = params
    sc_gather, sc_scatter = _sc_kernels()
    coords = coords.astype(_f32)
    masses = masses.astype(_f32)
    src = edge_index[0].astype(jnp.int32)
    dst = edge_index[1].astype(jnp.int32)
    ptr = batch_ptrs.astype(jnp.int32).reshape(1, B + 1)
    atoms2 = atoms.astype(jnp.int32).reshape(N, 1)

    def row(b):
        return b.astype(_f32).reshape(1, -1)

    Wn1, bn1, Wn2, bn2 = p['proj_node']
    Wc1, bc1, Wc2, bc2 = p['proj_cond']
    We1, be1, We2, be2 = p['proj_edge']
    Whn, bhn = p['head_norm']
    Wh1, bh1, Wh2, bh2 = p['head']

    st = (2.0 * np.pi / jnp.geomspace(0.001, 1.0, C // 2).astype(_f32)
          ).reshape(1, C // 2)
    sm = (2.0 * np.pi / jnp.geomspace(1e-4, 1e4, C // 2).astype(_f32)
          ).reshape(1, C // 2)

    sel3 = np.zeros((3, TD), np.float32)
    for j in range(3):
        sel3[j, 72 + j] = 1.0
    sel3 = jnp.asarray(sel3)
    rexp = np.zeros((H, F), np.float32)
    for j in range(H):
        rexp[j, 8 * j:8 * j + 8] = 1.0
    rexp = jnp.asarray(rexp)
    zpad = jnp.zeros((F, TD - 72), _f32)
    zedge = jnp.zeros((N, F), _f32)

    y = pl.pallas_call(
        _cond_body,
        out_shape=jax.ShapeDtypeStruct((B, C), _f32),
    )(t.astype(_f32), moments.astype(_f32), st, sm, Wc1, row(bc1), Wc2,
      row(bc2))

    h = _grid_call(
        _hembed_body, N // TN,
        [coords, atoms2, masses, p['emb_atom'], Wn1, row(bn1), Wn2, row(bn2)],
        [pl.BlockSpec((TN, 3), lambda i: (i, 0)),
         pl.BlockSpec((TN, 1), lambda i: (i, 0)),
         pl.BlockSpec((TN, 1), lambda i: (i, 0)),
         _full((90, 32)), _full(Wn1.shape), _full((1, bn1.shape[0])),
         _full(Wn2.shape), _full((1, bn2.shape[0]))],
        jax.ShapeDtypeStruct((N, F), _f32),
        pl.BlockSpec((TN, F), lambda i: (i, 0)))

    a = None
    for l in range(NL):
        bp = p['blocks'][l]
        first = (l == 0)
        last = (l == NL - 1)
        Wm, Wg = bp['Wm'], bp['Wg']
        wqs = jnp.concatenate(
            [jnp.concatenate([Wm[:F], Wg[:F]], axis=1), zpad], axis=1)
        wqd = jnp.concatenate(
            [jnp.concatenate([Wm[F:2 * F], Wg[F:2 * F]], axis=1), zpad],
            axis=1)
        wma, wga = Wm[2 * F:], Wg[2 * F:]

        ts, td, hn = _grid_call(
            _make_blocknode_body(first), N // TN,
            [h, y, ptr, coords, bp['Wc'], row(bp['bc']), wqs, wqd, sel3],
            [pl.BlockSpec((TN, F), lambda i: (i, 0)),
             _full((B, C)), _full((1, B + 1)),
             pl.BlockSpec((TN, 3), lambda i: (i, 0)),
             _full((F, 2 * F)), _full((1, 2 * F)),
             _full((F, TD)), _full((F, TD)), _full((3, TD))],
            (jax.ShapeDtypeStruct((N, TD), _f32),
             jax.ShapeDtypeStruct((N, TD), _f32),
             jax.ShapeDtypeStruct((N, F), _f32)),
            (pl.BlockSpec((TN, TD), lambda i: (i, 0)),
             pl.BlockSpec((TN, TD), lambda i: (i, 0)),
             pl.BlockSpec((TN, F), lambda i: (i, 0))))

        es, ed = sc_gather(ts, td, src, dst)

        a_in = a if a is not None else jnp.zeros((E, EF), _f32)
        edge_outs = _grid_call(
            _make_edge_body(first, last), E // TE,
            [es, ed, a_in, wma, row(bp['bm']), wga, row(bp['bg']), rexp,
             We1, row(be1), We2, row(be2),
             bp.get('Wa', jnp.zeros((F, EF), _f32)),
             row(bp.get('ba', jnp.zeros((EF,), _f32)))],
            [pl.BlockSpec((TE, TD), lambda i: (i, 0)),
             pl.BlockSpec((TE, TD), lambda i: (i, 0)),
             pl.BlockSpec((TE, EF), lambda i: (i, 0)),
             _full((EF, F)), _full((1, F)), _full((EF, H)), _full((1, H)),
             _full((H, F)), _full((3, 3)), _full((1, 3)), _full((3, EF)),
             _full((1, EF)), _full((F, EF)), _full((1, EF))],
            (jax.ShapeDtypeStruct((E, F), _f32),
             jax.ShapeDtypeStruct((E, EF), _f32)),
            (pl.BlockSpec((TE, F), lambda i: (i, 0)),
             pl.BlockSpec((TE, EF), lambda i: (i, 0))))
        m, anew = edge_outs
        if not last:
            a = anew

        partials = sc_scatter(m, dst, zedge)

        h = _grid_call(
            _update_body, N // TN,
            [h, hn, partials[0], partials[1], bp['Wu1'][:F], bp['Wu1'][F:],
             row(bp['bu1']), bp['Wu2'], row(bp['bu2'])],
            [pl.BlockSpec((TN, F), lambda i: (i, 0)),
             pl.BlockSpec((TN, F), lambda i: (i, 0)),
             pl.BlockSpec((TN, F), lambda i: (i, 0)),
             pl.BlockSpec((TN, F), lambda i: (i, 0)),
             _full((F, 4 * F)), _full((F, 4 * F)), _full((1, 4 * F)),
             _full((4 * F, F)), _full((1, F))],
            jax.ShapeDtypeStruct((N, F), _f32),
            pl.BlockSpec((TN, F), lambda i: (i, 0)))

    out = _grid_call(
        _head_body, N // TN,
        [h, y, ptr, coords, Whn, row(bhn), Wh1, row(bh1), Wh2, row(bh2)],
        [pl.BlockSpec((TN, F), lambda i: (i, 0)),
         _full((B, C)), _full((1, B + 1)),
         pl.BlockSpec((TN, 3), lambda i: (i, 0)),
         _full((C, 2 * F)), _full((1, 2 * F)),
         _full((F, 2 * F)), _full((1, 2 * F)),
         _full((2 * F, 3)), _full((1, 3))],
        jax.ShapeDtypeStruct((N, 3), _f32),
        pl.BlockSpec((TN, 3), lambda i: (i, 0)))
    return out
